# Initial kernel scaffold; baseline (speedup 1.0000x reference)
import jax
import jax.numpy as jnp
from jax.experimental import pallas as pl


def kernel(x, x_ref):
    raise NotImplementedError("write your pallas kernel here")


# speedup vs baseline: 78.0350x; 78.0350x over previous
import functools

import jax
import jax.numpy as jnp
from jax import lax
from jax.experimental import pallas as pl
from jax.experimental.pallas import tpu as pltpu
from jax.experimental.pallas import tpu_sc as plsc

N_GRID = 128
INV_DX = float(N_GRID)
P_MASS = (0.5 / N_GRID) ** 3
N_PART = 262144
NTOT = 2 * N_PART

NC = 2
NS = 16
L = 16

PER_TILE = NTOT // NS
GROUP = 1024
NGROUPS = PER_TILE // GROUP
NVEC = GROUP // L
LISTN = 27 * GROUP

HALF = N_GRID // NC
SLAB = N_GRID * N_GRID
HCELLS = HALF * SLAB
DUMMY = SLAB
GCELLS = HCELLS + DUMMY

ZPT = GCELLS // NS
ZCH = 4160
RPT = HCELLS // NS
RCH = 4096

_mesh = plsc.VectorSubcoreMesh(
    core_axis_name="c", subcore_axis_name="s", num_cores=NC, num_subcores=NS
)


@functools.partial(
    pl.kernel,
    out_type=jax.ShapeDtypeStruct((NC * NS, L), jnp.float32),
    mesh=_mesh,
    scratch_types=[
        pltpu.VMEM_SHARED((GCELLS,), jnp.float32),
        pltpu.VMEM((GROUP,), jnp.float32),
        pltpu.VMEM((GROUP,), jnp.float32),
        pltpu.VMEM((GROUP,), jnp.float32),
        pltpu.VMEM((LISTN,), jnp.int32),
        pltpu.VMEM((LISTN,), jnp.float32),
        pltpu.VMEM((ZCH,), jnp.float32),
        pltpu.VMEM((L,), jnp.float32),
    ],
)
def _p2g_loss(xs, ys, zs, out, grid, px, py, pz, idxl, vall, zbuf, accb):
    c = lax.axis_index("c")
    s = lax.axis_index("s")

    zero = jnp.zeros((L,), jnp.float32)

    @pl.loop(0, ZCH // L)
    def _(i):
        zbuf[pl.ds(i * L, L)] = zero

    @pl.loop(0, ZPT // ZCH)
    def _(i):
        pltpu.sync_copy(zbuf, grid.at[pl.ds(s * ZPT + i * ZCH, ZCH)])

    plsc.subcore_barrier()

    sign = jnp.where(s < NS // 2, jnp.float32(P_MASS), jnp.float32(-P_MASS))
    xoff = (-HALF) * c
    start = s * PER_TILE

    @pl.loop(0, NGROUPS)
    def _(g):
        gs = start + g * GROUP
        pltpu.sync_copy(xs.at[pl.ds(gs, GROUP)], px)
        pltpu.sync_copy(ys.at[pl.ds(gs, GROUP)], py)
        pltpu.sync_copy(zs.at[pl.ds(gs, GROUP)], pz)

        @pl.loop(0, NVEC)
        def _(b):
            off = b * L

            def basefx(p):
                t = p * INV_DX
                bi = (t - 0.5).astype(jnp.int32)
                return bi, t - bi.astype(jnp.float32)

            def wts(fx):
                return (
                    0.5 * (1.5 - fx) * (1.5 - fx),
                    0.75 - (fx - 1.0) * (fx - 1.0),
                    0.5 * (fx - 0.5) * (fx - 0.5),
                )

            bx, fxx = basefx(px[pl.ds(off, L)])
            by, fxy = basefx(py[pl.ds(off, L)])
            bz, fxz = basefx(pz[pl.ds(off, L)])
            wx = wts(fxx)
            wy = wts(fxy)
            wz = wts(fxz)
            lx = bx + xoff
            ybase = by * N_GRID
            yterm = (ybase, ybase + N_GRID, ybase + 2 * N_GRID)
            zterm = (bz, bz + 1, bz + 2)
            for i in range(3):
                lxi = lx + i
                ok = (lxi >= 0) & (lxi < HALF)
                xt = jnp.where(ok, lxi * SLAB, HCELLS)
                swi = wx[i] * sign
                for j in range(3):
                    idx_ij = xt + yterm[j]
                    w_ij = swi * wy[j]
                    for k in range(3):
                        pos = ((i * 3 + j) * 3 + k) * GROUP + off
                        idxl[pl.ds(pos, L)] = idx_ij + zterm[k]
                        vall[pl.ds(pos, L)] = w_ij * wz[k]

        pltpu.sync_copy(vall, grid.at[idxl], add=True)

    plsc.subcore_barrier()

    @pl.loop(0, RPT // RCH, init_carry=jnp.zeros((L,), jnp.float32))
    def acc(i, acc_o):
        pltpu.sync_copy(grid.at[pl.ds(s * RPT + i * RCH, RCH)], zbuf.at[pl.ds(0, RCH)])

        @pl.loop(0, RCH // L, init_carry=acc_o)
        def acc_i(j, a):
            return a + jnp.abs(zbuf[pl.ds(j * L, L)])

        return acc_i

    accb[...] = acc
    pltpu.sync_copy(accb, out.at[c * NS + s])


def kernel(x, x_ref):
    pts = jnp.concatenate([x, x_ref], axis=0)
    xs = pts[:, 0]
    ys = pts[:, 1]
    zs = pts[:, 2]
    partials = _p2g_loss(xs, ys, zs)
    return partials.sum()


# speedup vs baseline: 91.3165x; 1.1702x over previous
import functools

import jax
import jax.numpy as jnp
from jax import lax
from jax.experimental import pallas as pl
from jax.experimental.pallas import tpu as pltpu
from jax.experimental.pallas import tpu_sc as plsc

N_GRID = 128
INV_DX = float(N_GRID)
P_MASS = (0.5 / N_GRID) ** 3
N_PART = 262144
NTOT = 2 * N_PART

NC = 2
NS = 16
L = 16

PER_TILE = NTOT // NS
GROUP = 512
NGROUPS = PER_TILE // GROUP
NVEC = GROUP // L
LISTN = 27 * GROUP

HALF = N_GRID // NC
SLAB = N_GRID * N_GRID
HCELLS = HALF * SLAB
DUMMY = SLAB
GCELLS = HCELLS + DUMMY

ZPT = GCELLS // NS
ZCH = 4160
RPT = HCELLS // NS
RCH = 4096

_mesh = plsc.VectorSubcoreMesh(
    core_axis_name="c", subcore_axis_name="s", num_cores=NC, num_subcores=NS
)


@functools.partial(
    pl.kernel,
    out_type=jax.ShapeDtypeStruct((NC * NS, L), jnp.float32),
    mesh=_mesh,
    scratch_types=[
        pltpu.VMEM_SHARED((GCELLS,), jnp.float32),
        pltpu.VMEM((GROUP,), jnp.float32),
        pltpu.VMEM((GROUP,), jnp.float32),
        pltpu.VMEM((GROUP,), jnp.float32),
        pltpu.VMEM((LISTN,), jnp.int32),
        pltpu.VMEM((LISTN,), jnp.float32),
        pltpu.VMEM((LISTN,), jnp.int32),
        pltpu.VMEM((LISTN,), jnp.float32),
        pltpu.VMEM((ZCH,), jnp.float32),
        pltpu.VMEM((L,), jnp.float32),
        pltpu.SemaphoreType.DMA,
        pltpu.SemaphoreType.DMA,
    ],
)
def _p2g_loss(
    xs, ys, zs, out, grid, px, py, pz, idxl0, vall0, idxl1, vall1, zbuf, accb,
    sem0, sem1,
):
    c = lax.axis_index("c")
    s = lax.axis_index("s")

    zero = jnp.zeros((L,), jnp.float32)

    @pl.loop(0, ZCH // L)
    def _(i):
        zbuf[pl.ds(i * L, L)] = zero

    @pl.loop(0, ZPT // ZCH)
    def _(i):
        pltpu.sync_copy(zbuf, grid.at[pl.ds(s * ZPT + i * ZCH, ZCH)])

    plsc.subcore_barrier()

    sign = jnp.where(s < NS // 2, jnp.float32(P_MASS), jnp.float32(-P_MASS))
    xoff = (-HALF) * c
    start = s * PER_TILE

    slots = ((idxl0, vall0, sem0), (idxl1, vall1, sem1))

    @pl.loop(0, NGROUPS // 2)
    def _(gg):
        for par in range(2):
            idxl, vall, sem = slots[par]
            g = gg * 2 + par
            gs = start + g * GROUP
            pltpu.sync_copy(xs.at[pl.ds(gs, GROUP)], px)
            pltpu.sync_copy(ys.at[pl.ds(gs, GROUP)], py)
            pltpu.sync_copy(zs.at[pl.ds(gs, GROUP)], pz)

            @pl.when(gg > 0)
            def _():
                pltpu.make_async_copy(vall, grid.at[idxl], sem).wait()

            @pl.loop(0, NVEC)
            def _(b):
                off = b * L

                def basefx(p):
                    t = p * INV_DX
                    bi = (t - 0.5).astype(jnp.int32)
                    return bi, t - bi.astype(jnp.float32)

                def wts(fx):
                    return (
                        0.5 * (1.5 - fx) * (1.5 - fx),
                        0.75 - (fx - 1.0) * (fx - 1.0),
                        0.5 * (fx - 0.5) * (fx - 0.5),
                    )

                bx, fxx = basefx(px[pl.ds(off, L)])
                by, fxy = basefx(py[pl.ds(off, L)])
                bz, fxz = basefx(pz[pl.ds(off, L)])
                wx = wts(fxx)
                wy = wts(fxy)
                wz = wts(fxz)
                lx = bx + xoff
                ybase = by * N_GRID
                yterm = (ybase, ybase + N_GRID, ybase + 2 * N_GRID)
                zterm = (bz, bz + 1, bz + 2)
                for i in range(3):
                    lxi = lx + i
                    ok = (lxi >= 0) & (lxi < HALF)
                    xt = jnp.where(ok, lxi * SLAB, HCELLS)
                    swi = wx[i] * sign
                    for j in range(3):
                        idx_ij = xt + yterm[j]
                        w_ij = swi * wy[j]
                        for k in range(3):
                            pos = ((i * 3 + j) * 3 + k) * GROUP + off
                            idxl[pl.ds(pos, L)] = idx_ij + zterm[k]
                            vall[pl.ds(pos, L)] = w_ij * wz[k]

            pltpu.async_copy(vall, grid.at[idxl], sem, add=True)

    for idxl, vall, sem in slots:
        pltpu.make_async_copy(vall, grid.at[idxl], sem).wait()

    plsc.subcore_barrier()

    @pl.loop(0, RPT // RCH, init_carry=jnp.zeros((L,), jnp.float32))
    def acc(i, acc_o):
        pltpu.sync_copy(grid.at[pl.ds(s * RPT + i * RCH, RCH)], zbuf.at[pl.ds(0, RCH)])

        @pl.loop(0, RCH // L, init_carry=acc_o)
        def acc_i(j, a):
            return a + jnp.abs(zbuf[pl.ds(j * L, L)])

        return acc_i

    accb[...] = acc
    pltpu.sync_copy(accb, out.at[c * NS + s])


def kernel(x, x_ref):
    pts = jnp.concatenate([x, x_ref], axis=0)
    xs = pts[:, 0]
    ys = pts[:, 1]
    zs = pts[:, 2]
    partials = _p2g_loss(xs, ys, zs)
    return partials.sum()
